# baseline (speedup 1.0000x reference)
import functools

import jax
import jax.numpy as jnp
from jax import lax
from jax.experimental import pallas as pl
from jax.experimental.pallas import tpu as pltpu
from jax.experimental.pallas import tpu_sc as plsc

N = 10000
E = 320000
D = 128
C = 64
K = 8
ALPHA = 0.1
LAMBD = 1.0
C1 = LAMBD / (LAMBD + ALPHA)
C2 = ALPHA / (LAMBD + ALPHA)

CH = 32
NS = 16
NPAD = 10240
NR = NPAD // NS
RB = 80
NRB = NR // RB
CB = 128
CHUNKS = 160
EW = CHUNKS * CB
EPAD = NS * EW
GROUPS = CHUNKS // 4


def _mm_body(x_ref, w1_ref, b1_ref, w2_ref, b2_ref, o1_ref, o2_ref):
    x = x_ref[...]
    o1_ref[...] = jnp.dot(x, w1_ref[...], preferred_element_type=jnp.float32) + b1_ref[...]
    o2_ref[...] = jnp.dot(x, w2_ref[...], preferred_element_type=jnp.float32) + b2_ref[...]


def _mlp_affines(features, W1, b1, W2, b2):
    return pl.pallas_call(
        _mm_body,
        grid=(10,),
        in_specs=[
            pl.BlockSpec((1000, D), lambda i: (i, 0)),
            pl.BlockSpec((D, C), lambda i: (0, 0)),
            pl.BlockSpec((1, C), lambda i: (0, 0)),
            pl.BlockSpec((D, C), lambda i: (0, 0)),
            pl.BlockSpec((1, C), lambda i: (0, 0)),
        ],
        out_specs=[
            pl.BlockSpec((1000, C), lambda i: (i, 0)),
            pl.BlockSpec((1000, C), lambda i: (i, 0)),
        ],
        out_shape=[jax.ShapeDtypeStruct((N, C), jnp.float32)] * 2,
    )(features, W1, b1.reshape(1, C), W2, b2.reshape(1, C))


def _rsqrt16(x):
    xi = lax.bitcast_convert_type(x, jnp.int32)
    yi = jnp.int32(0x5F3759DF) - lax.shift_right_arithmetic(xi, 1)
    y = lax.bitcast_convert_type(yi, jnp.float32)
    for _ in range(3):
        y = y * (1.5 - 0.5 * x * y * y)
    return y


def _splat16(vec_ref, i):
    return plsc.load_gather(vec_ref, [jnp.zeros((16,), jnp.int32) + i])


def _sc_body(sd_r, i1_r, i2_r, out_r, p_r,
             acc, wv_sp, av_sp, dinv_sp, dout_sp, *rest):
    bufs = rest[0:16]
    onesb = bufs[0]
    accv, ivv, pvv, zbv = rest[16:20]
    dv0, dv1 = rest[20:22]
    idxb = rest[22:30]
    gsems = rest[30:34]
    ssems = rest[34:38]
    isems = rest[38:46]
    usems = rest[46:52]

    c = lax.axis_index("c")
    s = lax.axis_index("s")
    row0 = s * NR
    co = c * NPAD

    def _fill_ones(i, carry):
        for l in range(CH // 16):
            onesb[i, pl.ds(l * 16, 16)] = jnp.full((16,), 1.0, jnp.float32)
        return carry
    lax.fori_loop(0, CB, _fill_ones, 0)

    def _fill_zero(i, carry):
        for l in range(CH // 16):
            zbv[i, pl.ds(l * 16, 16)] = jnp.zeros((16,), jnp.float32)
        return carry
    lax.fori_loop(0, RB, _fill_zero, 0)

    for rb in range(NRB):
        r = row0 + rb * RB
        pltpu.sync_copy(zbv, acc.at[pl.ds(r, RB)])
    plsc.subcore_barrier()

    def _stage_idx(g, q):
        pltpu.sync_copy(sd_r.at[c, s, g], idxb[q])

    def _deg_pass(row):
        _stage_idx(0, 0)

        def _degb(t, carry):
            for st in (0, 1):
                g = 2 * t + st
                for b in range(4):
                    pltpu.async_copy(onesb, acc.at[idxb[st].at[row, b]], gsems[b], add=True)

                @pl.when(g + 1 < GROUPS)
                def _():
                    _stage_idx(g + 1, 1 - st)
                for b in range(4):
                    pltpu.make_async_copy(onesb, acc.at[idxb[st].at[row, b]], gsems[b]).wait()
            return carry
        lax.fori_loop(0, GROUPS // 2, _degb, 0)
        plsc.subcore_barrier()

    iota16 = lax.iota(jnp.int32, 16)
    zeros16 = jnp.zeros((16,), jnp.int32)

    _deg_pass(0)
    for rb in range(NRB):
        r = row0 + rb * RB
        pltpu.sync_copy(acc.at[pl.ds(r, RB)], accv)
        pltpu.sync_copy(zbv, acc.at[pl.ds(r, RB)])

        def _colo(i, carry):
            rows = iota16 + i * 16
            d_o = _rsqrt16(jnp.maximum(plsc.load_gather(accv, [rows, zeros16]), 1.0))
            dv0[pl.ds(i * 16, 16)] = d_o
            return carry
        lax.fori_loop(0, RB // 16, _colo, 0)
        pltpu.sync_copy(dv0, dout_sp.at[pl.ds(r, RB)])
    plsc.subcore_barrier()

    _deg_pass(2)
    for rb in range(NRB):
        r = row0 + rb * RB
        pltpu.sync_copy(acc.at[pl.ds(r, RB)], accv)
        pltpu.sync_copy(zbv, acc.at[pl.ds(r, RB)])
        pltpu.sync_copy(dout_sp.at[pl.ds(r, RB)], dv0)

        def _coli(i, carry):
            rows = iota16 + i * 16
            d_i = _rsqrt16(jnp.maximum(plsc.load_gather(accv, [rows, zeros16]), 1.0))
            dv1[pl.ds(i * 16, 16)] = d_i
            return carry
        lax.fori_loop(0, RB // 16, _coli, 0)
        pltpu.sync_copy(dv1, dinv_sp.at[pl.ds(r, RB)])

        def _vecs(i, carry):
            sl = pl.ds(i * 16, 16)
            d_o = dv0[sl]
            d_i = dv1[sl]
            dv1[sl] = C1 * d_i * d_o
            dv0[sl] = C2 * d_o
            return carry
        lax.fori_loop(0, RB // 16, _vecs, 0)
        pltpu.sync_copy(dv1, wv_sp.at[pl.ds(r, RB)])
        pltpu.sync_copy(dv0, av_sp.at[pl.ds(r, RB)])

        pltpu.sync_copy(i1_r.at[pl.ds(co + r, RB)], ivv)

        def _pinit(i, carry):
            a_s = _splat16(dv0, i) * (1.0 / C2)
            for l in range(CH // 16):
                sl = pl.ds(l * 16, 16)
                pvv[i, sl] = a_s * ivv[i, sl]
            return carry
        lax.fori_loop(0, RB, _pinit, 0)
        pltpu.sync_copy(pvv, p_r.at[pl.ds(co + r, RB)])
    plsc.subcore_barrier()

    def _stage_async(g, q):
        pltpu.async_copy(sd_r.at[c, s, g], idxb[q], isems[q])

    def _stage_wait(g, q):
        pltpu.make_async_copy(sd_r.at[c, s, g], idxb[q], isems[q]).wait()

    def _gather_issue(st, q):
        b0 = st * 4
        for b in range(4):
            pltpu.async_copy(p_r.at[idxb[q].at[1, b]], bufs[b0 + b], gsems[st])

    def _gather_wait(st, q):
        b0 = st * 4
        for b in range(4):
            pltpu.make_async_copy(p_r.at[idxb[q].at[1, b]], bufs[b0 + b], gsems[st]).wait()

    def _scatter_issue(st, q):
        b0 = st * 4
        for b in range(4):
            pltpu.async_copy(bufs[b0 + b], acc.at[idxb[q].at[2, b]], ssems[st], add=True)

    def _scatter_drain(st, q):
        b0 = st * 4
        for b in range(4):
            pltpu.make_async_copy(bufs[b0 + b], acc.at[idxb[q].at[2, b]], ssems[st]).wait()

    def _edge_phase():
        _stage_async(0, 0)
        _stage_async(1, 1)
        _stage_async(2, 2)
        _stage_wait(0, 0)
        _gather_issue(0, 0)
        _stage_wait(1, 1)
        _gather_issue(1, 1)

        def _eb(t, carry):
            g0 = 8 * t
            for j in range(8):
                g = g0 + j
                st = j % 4

                @pl.when(g + 3 < GROUPS)
                def _():
                    _stage_async(g + 3, (j + 3) % 8)
                _gather_wait(st, j)
                _scatter_issue(st, j)

                @pl.when(g >= 2)
                def _():
                    _scatter_drain((j + 2) % 4, (j + 6) % 8)

                @pl.when(g + 2 < GROUPS)
                def _():
                    _stage_wait(g + 2, (j + 2) % 8)
                    _gather_issue((j + 2) % 4, (j + 2) % 8)
            return carry
        lax.fori_loop(0, GROUPS // 8, _eb, 0)
        _scatter_drain((GROUPS - 2) % 4, (GROUPS - 2) % 8)
        _scatter_drain((GROUPS - 1) % 4, (GROUPS - 1) % 8)

    def _update_phase(anchor_r, last):
        for rb in range(NRB):
            r = row0 + rb * RB
            pltpu.async_copy(acc.at[pl.ds(r, RB)], accv, usems[0])
            pltpu.async_copy(anchor_r.at[pl.ds(co + r, RB)], ivv, usems[1])
            if last:
                pltpu.async_copy(dinv_sp.at[pl.ds(r, RB)], dv0, usems[2])
            else:
                pltpu.async_copy(wv_sp.at[pl.ds(r, RB)], dv0, usems[2])
                pltpu.async_copy(av_sp.at[pl.ds(r, RB)], dv1, usems[3])
            pltpu.make_async_copy(acc.at[pl.ds(r, RB)], accv, usems[0]).wait()
            pltpu.async_copy(zbv, acc.at[pl.ds(r, RB)], usems[4])
            pltpu.make_async_copy(anchor_r.at[pl.ds(co + r, RB)], ivv, usems[1]).wait()
            pltpu.make_async_copy(dinv_sp.at[pl.ds(r, RB)], dv0, usems[2]).wait()
            if not last:
                pltpu.make_async_copy(av_sp.at[pl.ds(r, RB)], dv1, usems[3]).wait()
            if rb > 0:
                pltpu.make_async_copy(pvv, p_r.at[pl.ds(co + row0, RB)], usems[5]).wait()

            if last:
                def _ub(i, carry):
                    w_s = C1 * _splat16(dv0, i)
                    for l in range(CH // 16):
                        sl = pl.ds(l * 16, 16)
                        pvv[i, sl] = w_s * accv[i, sl] + C2 * ivv[i, sl]
                    return carry
            else:
                def _ub(i, carry):
                    w_s = _splat16(dv0, i)
                    a_s = _splat16(dv1, i)
                    for l in range(CH // 16):
                        sl = pl.ds(l * 16, 16)
                        pvv[i, sl] = w_s * accv[i, sl] + a_s * ivv[i, sl]
                    return carry
            lax.fori_loop(0, RB, _ub, 0)
            if last:
                pltpu.async_copy(pvv, out_r.at[pl.ds(co + r, RB)], usems[5])
            else:
                pltpu.async_copy(pvv, p_r.at[pl.ds(co + r, RB)], usems[5])
        pltpu.make_async_copy(pvv, p_r.at[pl.ds(co + row0, RB)], usems[5]).wait()
        for _ in range(NRB):
            pltpu.make_async_copy(zbv, acc.at[pl.ds(row0, RB)], usems[4]).wait()

    def _rounds(anchor_r, nsteps):
        def _sb(k, carry):
            _edge_phase()
            plsc.subcore_barrier()
            _update_phase(anchor_r, last=False)
            plsc.subcore_barrier()
            return carry
        lax.fori_loop(0, nsteps, _sb, 0)

    _rounds(i1_r, K)
    _rounds(i2_r, K - 1)
    _edge_phase()
    plsc.subcore_barrier()
    _update_phase(i2_r, last=True)


_sc_kernel = functools.partial(
    pl.kernel,
    out_type=(
        jax.ShapeDtypeStruct((2 * NPAD, CH), jnp.float32),
        jax.ShapeDtypeStruct((2 * NPAD, CH), jnp.float32),
    ),
    mesh=plsc.VectorSubcoreMesh(core_axis_name="c", subcore_axis_name="s"),
    compiler_params=pltpu.CompilerParams(
        use_tc_tiling_on_sc=False, needs_layout_passes=False),
    scratch_types=(
        [
            pltpu.VMEM_SHARED((NPAD, CH), jnp.float32),
            pltpu.VMEM_SHARED((NPAD,), jnp.float32),
            pltpu.VMEM_SHARED((NPAD,), jnp.float32),
            pltpu.VMEM_SHARED((NPAD,), jnp.float32),
            pltpu.VMEM_SHARED((NPAD,), jnp.float32),
        ]
        + [pltpu.VMEM((CB, CH), jnp.float32) for _ in range(16)]
        + [pltpu.VMEM((RB, CH), jnp.float32) for _ in range(4)]
        + [pltpu.VMEM((RB,), jnp.float32) for _ in range(2)]
        + [pltpu.VMEM((3, 4, CB), jnp.int32) for _ in range(8)]
        + [pltpu.SemaphoreType.DMA for _ in range(22)]
    ),
)(_sc_body)


def kernel(features, edge_index, W1, b1, W2, b2):
    o1, o2 = _mlp_affines(features, W1, b1, W2, b2)
    i1p = jnp.pad(o1, ((0, NPAD - N), (0, 0)))
    i2p = jnp.pad(o2, ((0, NPAD - N), (0, 0)))
    i1f = jnp.concatenate([i1p[:, :CH], i1p[:, CH:]], axis=0)
    i2f = jnp.concatenate([i2p[:, :CH], i2p[:, CH:]], axis=0)
    srcp = jnp.pad(edge_index[0], (0, EPAD - E), constant_values=N).reshape(NS, GROUPS, 4, CB)
    dstp = jnp.pad(edge_index[1], (0, EPAD - E), constant_values=N).reshape(NS, GROUPS, 4, CB)
    sd0 = jnp.stack([srcp, srcp, dstp], axis=2)
    sd1 = jnp.stack([srcp, srcp + NPAD, dstp], axis=2)
    sd = jnp.stack([sd0, sd1])
    outf, _ = _sc_kernel(sd, i1f, i2f)
    o = outf.reshape(2, NPAD, CH)
    return jnp.concatenate([o[0, :N], o[1, :N]], axis=1)


# speedup vs baseline: 2.0670x; 2.0670x over previous
import functools

import jax
import jax.numpy as jnp
from jax import lax
from jax.experimental import pallas as pl
from jax.experimental.pallas import tpu as pltpu
from jax.experimental.pallas import tpu_sc as plsc

N = 10000
E = 320000
D = 128
C = 64
K = 8
ALPHA = 0.1
LAMBD = 1.0
C1 = LAMBD / (LAMBD + ALPHA)
C2 = ALPHA / (LAMBD + ALPHA)

CH = 32
NS = 16
NPAD = 10240
NR = NPAD // NS
RB = 64
NRB = NR // RB
CB = 128
CHUNKS = 160
EW = CHUNKS * CB
EPAD = NS * EW
GROUPS = CHUNKS // 4


def _mm_body(x_ref, w1_ref, b1_ref, w2_ref, b2_ref, o1_ref, o2_ref):
    x = x_ref[...]
    o1_ref[...] = jnp.dot(x, w1_ref[...], preferred_element_type=jnp.float32) + b1_ref[...]
    o2_ref[...] = jnp.dot(x, w2_ref[...], preferred_element_type=jnp.float32) + b2_ref[...]


def _mlp_affines(features, W1, b1, W2, b2):
    return pl.pallas_call(
        _mm_body,
        grid=(10,),
        in_specs=[
            pl.BlockSpec((1000, D), lambda i: (i, 0)),
            pl.BlockSpec((D, C), lambda i: (0, 0)),
            pl.BlockSpec((1, C), lambda i: (0, 0)),
            pl.BlockSpec((D, C), lambda i: (0, 0)),
            pl.BlockSpec((1, C), lambda i: (0, 0)),
        ],
        out_specs=[
            pl.BlockSpec((1000, C), lambda i: (i, 0)),
            pl.BlockSpec((1000, C), lambda i: (i, 0)),
        ],
        out_shape=[jax.ShapeDtypeStruct((N, C), jnp.float32)] * 2,
    )(features, W1, b1.reshape(1, C), W2, b2.reshape(1, C))


def _rsqrt16(x):
    xi = lax.bitcast_convert_type(x, jnp.int32)
    yi = jnp.int32(0x5F3759DF) - lax.shift_right_arithmetic(xi, 1)
    y = lax.bitcast_convert_type(yi, jnp.float32)
    for _ in range(3):
        y = y * (1.5 - 0.5 * x * y * y)
    return y


def _splat16(vec_ref, i):
    return plsc.load_gather(vec_ref, [jnp.zeros((16,), jnp.int32) + i])


def _sc_body(sd_r, i1_r, i2_r, out_r,
             acc, psp, wv_sp, av_sp, dinv_sp, *rest):
    bufs = rest[0:16]
    onesb = bufs[0]
    accv, ivv, pvv, zbv = rest[16:20]
    dv0, dv1 = rest[20:22]
    accv2, ivv2 = rest[22:24]
    dv2, dv3 = rest[24:26]
    idxb = rest[26:34]
    gsems = rest[34:38]
    ssems = rest[38:42]
    isems = rest[42:50]
    usems = rest[50:56]

    c = lax.axis_index("c")
    s = lax.axis_index("s")
    row0 = s * NR
    co = c * NPAD

    def _fill_ones(i, carry):
        for l in range(CH // 16):
            onesb[i, pl.ds(l * 16, 16)] = jnp.full((16,), 1.0, jnp.float32)
        return carry
    lax.fori_loop(0, CB, _fill_ones, 0)

    def _fill_zero(i, carry):
        for l in range(CH // 16):
            zbv[i, pl.ds(l * 16, 16)] = jnp.zeros((16,), jnp.float32)
        return carry
    lax.fori_loop(0, RB, _fill_zero, 0)

    for rb in range(NRB):
        r = row0 + rb * RB
        pltpu.sync_copy(zbv, acc.at[pl.ds(r, RB)])
        pltpu.sync_copy(zbv, psp.at[pl.ds(r, RB)])
    plsc.subcore_barrier()

    def _stage_idx(g, q):
        pltpu.sync_copy(sd_r.at[s, g], idxb[q])

    _stage_idx(0, 0)

    def _degb(t, carry):
        for st in (0, 1):
            g = 2 * t + st
            for b in range(4):
                pltpu.async_copy(onesb, acc.at[idxb[st].at[0, b]], gsems[b], add=True)
                pltpu.async_copy(onesb, psp.at[idxb[st].at[1, b]], ssems[b], add=True)

            @pl.when(g + 1 < GROUPS)
            def _():
                _stage_idx(g + 1, 1 - st)
            for b in range(4):
                pltpu.make_async_copy(onesb, acc.at[idxb[st].at[0, b]], gsems[b]).wait()
                pltpu.make_async_copy(onesb, psp.at[idxb[st].at[1, b]], ssems[b]).wait()
        return carry
    lax.fori_loop(0, GROUPS // 2, _degb, 0)
    plsc.subcore_barrier()

    iota16 = lax.iota(jnp.int32, 16)
    zeros16 = jnp.zeros((16,), jnp.int32)
    for rb in range(NRB):
        r = row0 + rb * RB
        pltpu.sync_copy(acc.at[pl.ds(r, RB)], accv)
        pltpu.sync_copy(psp.at[pl.ds(r, RB)], ivv)

        def _col(i, carry):
            rows = iota16 + i * 16
            d_o = _rsqrt16(jnp.maximum(plsc.load_gather(accv, [rows, zeros16]), 1.0))
            d_i = _rsqrt16(jnp.maximum(plsc.load_gather(ivv, [rows, zeros16]), 1.0))
            dv0[pl.ds(i * 16, 16)] = d_o
            dv1[pl.ds(i * 16, 16)] = d_i
            return carry
        lax.fori_loop(0, RB // 16, _col, 0)

        def _vecs(i, carry):
            sl = pl.ds(i * 16, 16)
            d_o = dv0[sl]
            d_i = dv1[sl]
            dv1[sl] = C1 * d_i * d_o
            dv0[sl] = C2 * d_o
            return carry
        pltpu.sync_copy(dv1, dinv_sp.at[pl.ds(r, RB)])
        lax.fori_loop(0, RB // 16, _vecs, 0)
        pltpu.sync_copy(dv1, wv_sp.at[pl.ds(r, RB)])
        pltpu.sync_copy(dv0, av_sp.at[pl.ds(r, RB)])

        pltpu.sync_copy(i1_r.at[pl.ds(co + r, RB)], ivv)
        pltpu.sync_copy(zbv, acc.at[pl.ds(r, RB)])

        def _pinit(i, carry):
            a_s = _splat16(dv0, i) * (1.0 / C2)
            for l in range(CH // 16):
                sl = pl.ds(l * 16, 16)
                pvv[i, sl] = a_s * ivv[i, sl]
            return carry
        lax.fori_loop(0, RB, _pinit, 0)
        pltpu.sync_copy(pvv, psp.at[pl.ds(r, RB)])
    plsc.subcore_barrier()

    def _stage_async(g, q):
        pltpu.async_copy(sd_r.at[s, g], idxb[q], isems[q])

    def _stage_wait(g, q):
        pltpu.make_async_copy(sd_r.at[s, g], idxb[q], isems[q]).wait()

    def _gather_issue(st, q):
        b0 = st * 4
        for b in range(4):
            pltpu.async_copy(psp.at[idxb[q].at[0, b]], bufs[b0 + b], gsems[st])

    def _gather_wait(st, q):
        b0 = st * 4
        for b in range(4):
            pltpu.make_async_copy(psp.at[idxb[q].at[0, b]], bufs[b0 + b], gsems[st]).wait()

    def _scatter_issue(st, q):
        b0 = st * 4
        for b in range(4):
            pltpu.async_copy(bufs[b0 + b], acc.at[idxb[q].at[1, b]], ssems[st], add=True)

    def _scatter_drain(st, q):
        b0 = st * 4
        for b in range(4):
            pltpu.make_async_copy(bufs[b0 + b], acc.at[idxb[q].at[1, b]], ssems[st]).wait()

    def _edge_phase():
        _stage_async(0, 0)
        _stage_async(1, 1)
        _stage_async(2, 2)
        _stage_wait(0, 0)
        _gather_issue(0, 0)
        _stage_wait(1, 1)
        _gather_issue(1, 1)

        def _eb(t, carry):
            g0 = 8 * t
            for j in range(8):
                g = g0 + j
                st = j % 4

                @pl.when(g + 3 < GROUPS)
                def _():
                    _stage_async(g + 3, (j + 3) % 8)
                _gather_wait(st, j)
                _scatter_issue(st, j)

                @pl.when(g >= 2)
                def _():
                    _scatter_drain((j + 2) % 4, (j + 6) % 8)

                @pl.when(g + 2 < GROUPS)
                def _():
                    _stage_wait(g + 2, (j + 2) % 8)
                    _gather_issue((j + 2) % 4, (j + 2) % 8)
            return carry
        lax.fori_loop(0, GROUPS // 8, _eb, 0)
        _scatter_drain((GROUPS - 2) % 4, (GROUPS - 2) % 8)
        _scatter_drain((GROUPS - 1) % 4, (GROUPS - 1) % 8)

    def _update_phase(anchor_r, last):
        sets = ((accv, ivv, dv0, dv1), (accv2, ivv2, dv2, dv3))

        def _fire(rb, bs):
            av_, iv_, d0, d1 = bs
            r = row0 + rb * RB
            pltpu.async_copy(acc.at[pl.ds(r, RB)], av_, usems[0])
            pltpu.async_copy(anchor_r.at[pl.ds(co + r, RB)], iv_, usems[1])
            if last:
                pltpu.async_copy(dinv_sp.at[pl.ds(r, RB)], d0, usems[2])
            else:
                pltpu.async_copy(wv_sp.at[pl.ds(r, RB)], d0, usems[2])
                pltpu.async_copy(av_sp.at[pl.ds(r, RB)], d1, usems[3])

        def _wait_in(rb, bs):
            av_, iv_, d0, d1 = bs
            r = row0 + rb * RB
            pltpu.make_async_copy(acc.at[pl.ds(r, RB)], av_, usems[0]).wait()
            pltpu.async_copy(zbv, acc.at[pl.ds(r, RB)], usems[4])
            pltpu.make_async_copy(anchor_r.at[pl.ds(co + r, RB)], iv_, usems[1]).wait()
            pltpu.make_async_copy(dinv_sp.at[pl.ds(r, RB)], d0, usems[2]).wait()
            if not last:
                pltpu.make_async_copy(av_sp.at[pl.ds(r, RB)], d1, usems[3]).wait()

        _fire(0, sets[0])
        for rb in range(NRB):
            bs = sets[rb % 2]
            av_, iv_, d0, d1 = bs
            r = row0 + rb * RB
            _wait_in(rb, bs)
            if rb + 1 < NRB:
                _fire(rb + 1, sets[(rb + 1) % 2])
            if rb > 0:
                pltpu.make_async_copy(pvv, psp.at[pl.ds(row0, RB)], usems[5]).wait()

            if last:
                def _ub(i, carry):
                    w_s = C1 * _splat16(d0, i)
                    for l in range(CH // 16):
                        sl = pl.ds(l * 16, 16)
                        pvv[i, sl] = w_s * av_[i, sl] + C2 * iv_[i, sl]
                    return carry
            else:
                def _ub(i, carry):
                    w_s = _splat16(d0, i)
                    a_s = _splat16(d1, i)
                    for l in range(CH // 16):
                        sl = pl.ds(l * 16, 16)
                        pvv[i, sl] = w_s * av_[i, sl] + a_s * iv_[i, sl]
                    return carry
            lax.fori_loop(0, RB, _ub, 0)
            if last:
                pltpu.async_copy(pvv, out_r.at[pl.ds(co + r, RB)], usems[5])
            else:
                pltpu.async_copy(pvv, psp.at[pl.ds(r, RB)], usems[5])
        pltpu.make_async_copy(pvv, psp.at[pl.ds(row0, RB)], usems[5]).wait()
        for _ in range(NRB):
            pltpu.make_async_copy(zbv, acc.at[pl.ds(row0, RB)], usems[4]).wait()

    def _rounds(anchor_r, nsteps):
        def _sb(k, carry):
            _edge_phase()
            plsc.subcore_barrier()
            _update_phase(anchor_r, last=False)
            plsc.subcore_barrier()
            return carry
        lax.fori_loop(0, nsteps, _sb, 0)

    _rounds(i1_r, K)
    _rounds(i2_r, K - 1)
    _edge_phase()
    plsc.subcore_barrier()
    _update_phase(i2_r, last=True)


_sc_kernel = functools.partial(
    pl.kernel,
    out_type=jax.ShapeDtypeStruct((2 * NPAD, CH), jnp.float32),
    mesh=plsc.VectorSubcoreMesh(core_axis_name="c", subcore_axis_name="s"),
    compiler_params=pltpu.CompilerParams(
        use_tc_tiling_on_sc=False, needs_layout_passes=False),
    scratch_types=(
        [
            pltpu.VMEM_SHARED((NPAD, CH), jnp.float32),
            pltpu.VMEM_SHARED((NPAD, CH), jnp.float32),
            pltpu.VMEM_SHARED((NPAD,), jnp.float32),
            pltpu.VMEM_SHARED((NPAD,), jnp.float32),
            pltpu.VMEM_SHARED((NPAD,), jnp.float32),
        ]
        + [pltpu.VMEM((CB, CH), jnp.float32) for _ in range(16)]
        + [pltpu.VMEM((RB, CH), jnp.float32) for _ in range(4)]
        + [pltpu.VMEM((RB,), jnp.float32) for _ in range(2)]
        + [pltpu.VMEM((RB, CH), jnp.float32) for _ in range(2)]
        + [pltpu.VMEM((RB,), jnp.float32) for _ in range(2)]
        + [pltpu.VMEM((2, 4, CB), jnp.int32) for _ in range(8)]
        + [pltpu.SemaphoreType.DMA for _ in range(22)]
    ),
)(_sc_body)


def kernel(features, edge_index, W1, b1, W2, b2):
    o1, o2 = _mlp_affines(features, W1, b1, W2, b2)
    i1p = jnp.pad(o1, ((0, NPAD - N), (0, 0)))
    i2p = jnp.pad(o2, ((0, NPAD - N), (0, 0)))
    i1f = jnp.concatenate([i1p[:, :CH], i1p[:, CH:]], axis=0)
    i2f = jnp.concatenate([i2p[:, :CH], i2p[:, CH:]], axis=0)
    srcp = jnp.pad(edge_index[0], (0, EPAD - E), constant_values=N).reshape(NS, GROUPS, 4, CB)
    dstp = jnp.pad(edge_index[1], (0, EPAD - E), constant_values=N).reshape(NS, GROUPS, 4, CB)
    sd = jnp.stack([srcp, dstp], axis=2)
    outf = _sc_kernel(sd, i1f, i2f)
    o = outf.reshape(2, NPAD, CH)
    return jnp.concatenate([o[0, :N], o[1, :N]], axis=1)
